# baseline (speedup 1.0000x reference)
import functools

import jax
import jax.numpy as jnp
from jax import lax
from jax.experimental import pallas as pl
from jax.experimental.pallas import tpu as pltpu
from jax.experimental.pallas import tpu_sc as plsc

V = 10000
E = 5000
NNZ = 320000
D = 128

NC = 2
NS = 16
NW = NC * NS
PER_W = NNZ // NW
G = 80
NBLK = PER_W // G
EP = 5120
VP = 10240
EP_T = EP // NS
VP_T = VP // NS
YREP = 16

f32 = jnp.float32


def _sc_mesh():
    return plsc.VectorSubcoreMesh(core_axis_name="c", subcore_axis_name="s")


def _hist_body(src_hbm, dst_hbm, zv_hbm, ze_hbm, outv, oute,
               idx_s, idx_d, ones_v, hv_sh, he_sh):
    cid = lax.axis_index("c")
    sid = lax.axis_index("s")
    wid = cid * NS + sid
    pltpu.sync_copy(zv_hbm, hv_sh.at[pl.ds(sid * VP_T, VP_T)])

    @pl.when(sid == 0)
    def _():
        pltpu.sync_copy(ze_hbm, he_sh)

    pltpu.sync_copy(src_hbm.at[wid], idx_s)
    pltpu.sync_copy(dst_hbm.at[wid], idx_d)
    for k in range(G // 16):
        ones_v[pl.ds(k * 16, 16)] = jnp.full((16,), 1.0, f32)
    plsc.subcore_barrier()

    @pl.loop(0, NBLK)
    def _(g):
        pltpu.sync_copy(ones_v, hv_sh.at[idx_s.at[g]], add=True)
        pltpu.sync_copy(ones_v, he_sh.at[idx_d.at[g]], add=True)

    plsc.subcore_barrier()
    pltpu.sync_copy(hv_sh.at[pl.ds(sid * VP_T, VP_T)],
                    outv.at[pl.ds(cid * VP + sid * VP_T, VP_T)])

    @pl.when(sid == 0)
    def _():
        pltpu.sync_copy(he_sh, oute.at[pl.ds(cid * EP, EP)])


def _hist(src3, dst3, zv, ze):
    k = pl.kernel(
        _hist_body,
        out_type=(jax.ShapeDtypeStruct((NC * VP,), f32),
                  jax.ShapeDtypeStruct((NC * EP,), f32)),
        mesh=_sc_mesh(),
        scratch_types=[
            pltpu.VMEM((NBLK, G), jnp.int32),
            pltpu.VMEM((NBLK, G), jnp.int32),
            pltpu.VMEM((G,), f32),
            pltpu.VMEM_SHARED((VP,), f32),
            pltpu.VMEM_SHARED((EP,), f32),
        ],
    )
    return k(src3, dst3, zv, ze)


def _v2e_body(xt_hbm, src_hbm, dst_hbm, dv_hbm, zr_hbm, zs_hbm,
              outy, outs,
              idx_s, idx_d, rows_a, rows_b, vals_a, vals_b, y_sh, s_sh,
              sg, sv):
    cid = lax.axis_index("c")
    sid = lax.axis_index("s")
    wid = cid * NS + sid
    pltpu.sync_copy(zr_hbm, y_sh.at[pl.ds(sid * EP_T, EP_T)])

    @pl.when(sid == 0)
    def _():
        pltpu.sync_copy(zs_hbm, s_sh)

    pltpu.sync_copy(src_hbm.at[wid], idx_s)
    pltpu.sync_copy(dst_hbm.at[wid], idx_d)
    pltpu.async_copy(xt_hbm.at[idx_s.at[0]], rows_a, sg)
    pltpu.async_copy(dv_hbm.at[idx_s.at[0]], vals_a, sv)
    plsc.subcore_barrier()

    def step(g, rcur, vcur, rnxt, vnxt, last):
        pltpu.make_async_copy(xt_hbm.at[idx_s.at[g]], rcur, sg).wait()
        pltpu.make_async_copy(dv_hbm.at[idx_s.at[g]], vcur, sv).wait()
        if not last:
            pltpu.async_copy(xt_hbm.at[idx_s.at[g + 1]], rnxt, sg)
            pltpu.async_copy(dv_hbm.at[idx_s.at[g + 1]], vnxt, sv)
        pltpu.sync_copy(rcur, y_sh.at[idx_d.at[g]], add=True)
        pltpu.sync_copy(vcur, s_sh.at[idx_d.at[g]], add=True)

    @pl.loop(0, (NB2 - 2) // 2)
    def _(p):
        step(2 * p, rows_a, vals_a, rows_b, vals_b, False)
        step(2 * p + 1, rows_b, vals_b, rows_a, vals_a, False)

    step(NB2 - 2, rows_a, vals_a, rows_b, vals_b, False)
    step(NB2 - 1, rows_b, vals_b, None, None, True)

    plsc.subcore_barrier()
    pltpu.sync_copy(y_sh.at[pl.ds(sid * EP_T, EP_T)],
                    outy.at[cid, pl.ds(sid * EP_T, EP_T)])

    @pl.when(sid == 0)
    def _():
        pltpu.sync_copy(s_sh, outs.at[pl.ds(cid * EP, EP)])


def _v2e(xt, src3, dst3, dv1, zr, zs):
    k = pl.kernel(
        _v2e_body,
        out_type=(jax.ShapeDtypeStruct((NC, EP, D), f32),
                  jax.ShapeDtypeStruct((NC * EP,), f32)),
        mesh=_sc_mesh(),
        scratch_types=[
            pltpu.VMEM((NB2, G2), jnp.int32),
            pltpu.VMEM((NB2, G2), jnp.int32),
            pltpu.VMEM((G2, D), f32),
            pltpu.VMEM((G2, D), f32),
            pltpu.VMEM((G2,), f32),
            pltpu.VMEM((G2,), f32),
            pltpu.VMEM_SHARED((EP, D), f32),
            pltpu.VMEM_SHARED((EP,), f32),
            pltpu.SemaphoreType.DMA,
            pltpu.SemaphoreType.DMA,
        ],
    )
    return k(xt, src3, dst3, dv1, zr, zs)


YS = 5120
YS_T = YS // NS


G2 = 128
NNZP = NW * VP
NB2 = VP // G2
NCH = 8
CB = 10


def _e2v_body(y_hbm, src_hbm, dst_hbm, zr_hbm, outx,
              idx_s, idx_d, rows_a, rows_b, x_sh, sg):
    cid = lax.axis_index("c")
    sid = lax.axis_index("s")
    wid = cid * NS + sid
    pltpu.sync_copy(zr_hbm, x_sh.at[pl.ds(sid * VP_T, VP_T)])
    plsc.subcore_barrier()

    @pl.loop(0, NCH)
    def _(c):
        pltpu.sync_copy(src_hbm.at[wid, c], idx_s)
        pltpu.sync_copy(dst_hbm.at[wid, c], idx_d)
        pltpu.async_copy(y_hbm.at[idx_d.at[0]], rows_a, sg)

        def step(g, rcur, rnxt, last):
            pltpu.make_async_copy(y_hbm.at[idx_d.at[g]], rcur, sg).wait()
            if not last:
                pltpu.async_copy(y_hbm.at[idx_d.at[g + 1]], rnxt, sg)
            pltpu.sync_copy(rcur, x_sh.at[idx_s.at[g]], add=True)

        @pl.loop(0, (CB - 2) // 2)
        def _(p):
            step(2 * p, rows_a, rows_b, False)
            step(2 * p + 1, rows_b, rows_a, False)

        step(CB - 2, rows_a, rows_b, False)
        step(CB - 1, rows_b, None, True)

    plsc.subcore_barrier()
    pltpu.sync_copy(x_sh.at[pl.ds(sid * VP_T, VP_T)],
                    outx.at[cid, pl.ds(sid * VP_T, VP_T)])


def _e2v(ynorm, src4, dst4, zrv):
    k = pl.kernel(
        _e2v_body,
        out_type=jax.ShapeDtypeStruct((NC, VP, D), f32),
        mesh=_sc_mesh(),
        scratch_types=[
            pltpu.VMEM((CB, G2), jnp.int32),
            pltpu.VMEM((CB, G2), jnp.int32),
            pltpu.VMEM((G2, D), f32),
            pltpu.VMEM((G2, D), f32),
            pltpu.VMEM_SHARED((VP, D), f32),
            pltpu.SemaphoreType.DMA,
        ],
    )
    return k(ynorm, src4, dst4, zrv)


def _matmul_body(x_ref, w_ref, b_ref, o_ref):
    o_ref[...] = lax.dot_general(
        x_ref[...], w_ref[...], (((1,), (1,)), ((), ())),
        preferred_element_type=f32) + b_ref[...]


def _matmul(X, W, b):
    return pl.pallas_call(
        _matmul_body,
        grid=(10,),
        in_specs=[
            pl.BlockSpec((V // 10, D), lambda i: (i, 0)),
            pl.BlockSpec((D, D), lambda i: (0, 0)),
            pl.BlockSpec((1, D), lambda i: (0, 0)),
        ],
        out_specs=pl.BlockSpec((V // 10, D), lambda i: (i, 0)),
        out_shape=jax.ShapeDtypeStruct((V, D), f32),
    )(X, W, b.reshape(1, D))


def _combine_body(d0, d1, c0, c1, dv, dvn, cnt):
    dsum = d0[...] + d1[...]
    dv[...] = dsum
    dvn[...] = jnp.where(dsum > 0, lax.rsqrt(jnp.maximum(dsum, 1e-12)), 0.0)
    cnt[...] = c0[...] + c1[...]


def _combine(d0, d1, c0, c1):
    return pl.pallas_call(
        _combine_body,
        out_shape=(jax.ShapeDtypeStruct((VP // D, D), f32),
                   jax.ShapeDtypeStruct((VP // D, D), f32),
                   jax.ShapeDtypeStruct((EP // D, D), f32)),
    )(d0, d1, c0, c1)


def _norm_body(y0, y1, s0, s1, c, out):
    cc = c[...]
    y = (y0[...] + y1[...]) / jnp.maximum(cc, 1.0)
    de = (s0[...] + s1[...]) / (cc + 1.0)
    fac = jnp.where(cc > 0, lax.rsqrt(jnp.maximum(de, 1e-12)), 1.0)
    out[...] = y * fac


def _norm(y0, y1, s0, s1, cnt):
    nb = 8
    return pl.pallas_call(
        _norm_body,
        grid=(nb * YREP,),
        in_specs=[
            pl.BlockSpec((EP // nb, D), lambda i: (i % nb, 0)),
            pl.BlockSpec((EP // nb, D), lambda i: (i % nb, 0)),
            pl.BlockSpec((EP // nb, 1), lambda i: (i % nb, 0)),
            pl.BlockSpec((EP // nb, 1), lambda i: (i % nb, 0)),
            pl.BlockSpec((EP // nb, 1), lambda i: (i % nb, 0)),
        ],
        out_specs=pl.BlockSpec((EP // nb, D), lambda i: (i, 0)),
        out_shape=jax.ShapeDtypeStruct((YREP * EP, D), f32),
    )(y0, y1, s0, s1, cnt)


def _final_body(x0, x1, dvn, out):
    out[...] = jnp.maximum(dvn[...] * (x0[...] + x1[...]), 0.0)


def _final(x0, x1, dvn):
    nb = 10
    return pl.pallas_call(
        _final_body,
        grid=(nb,),
        in_specs=[
            pl.BlockSpec((V // nb, D), lambda i: (i, 0)),
            pl.BlockSpec((V // nb, D), lambda i: (i, 0)),
            pl.BlockSpec((V // nb, 1), lambda i: (i, 0)),
        ],
        out_specs=pl.BlockSpec((V // nb, D), lambda i: (i, 0)),
        out_shape=jax.ShapeDtypeStruct((V, D), f32),
    )(x0, x1, dvn)


def kernel(X, v2e_src, v2e_dst, W, b):
    src3 = v2e_src.reshape(NW, NBLK, G)
    dst3 = v2e_dst.reshape(NW, NBLK, G)

    xt = _matmul(X, W, b)

    zv = jnp.zeros((VP_T,), f32)
    ze = jnp.zeros((EP,), f32)
    dvp, cep = _hist(src3, dst3, zv, ze)
    dvp = dvp.reshape(NC, VP)
    cep = cep.reshape(NC, EP)

    dv, dvneg, cnt = _combine(
        dvp[0].reshape(VP // D, D), dvp[1].reshape(VP // D, D),
        cep[0].reshape(EP // D, D), cep[1].reshape(EP // D, D))
    dv1 = dv.reshape(VP)

    npad = NNZP - NNZ
    iota_p = jnp.arange(npad, dtype=jnp.int32)
    srcp2 = jnp.concatenate([v2e_src, iota_p % V]).reshape(NW, NB2, G2)
    dstp2 = jnp.concatenate([v2e_dst, (iota_p % (EP - E)) + E]).reshape(
        NW, NB2, G2)
    zr = jnp.zeros((EP_T, D), f32)
    zs = jnp.zeros((EP,), f32)
    yp, sp = _v2e(xt, srcp2, dstp2, dv1, zr, zs)
    sp = sp.reshape(NC, EP)

    ynorm = _norm(yp[0], yp[1],
                  sp[0].reshape(EP, 1), sp[1].reshape(EP, 1),
                  cnt.reshape(EP, 1))

    srcp = jnp.concatenate([v2e_src, (iota_p % (VP - V)) + V])
    dstp = jnp.concatenate([v2e_dst, (iota_p % (EP - E)) + E])
    dstp = dstp + (jnp.arange(NNZP, dtype=jnp.int32) % YREP) * EP
    zrv = jnp.zeros((VP_T, D), f32)
    xp = _e2v(ynorm, srcp.reshape(NW, NCH, CB, G2),
              dstp.reshape(NW, NCH, CB, G2), zrv)

    return _final(xp[0], xp[1], dvneg.reshape(VP, 1)[:V])


# speedup vs baseline: 1.0795x; 1.0795x over previous
import functools

import jax
import jax.numpy as jnp
from jax import lax
from jax.experimental import pallas as pl
from jax.experimental.pallas import tpu as pltpu
from jax.experimental.pallas import tpu_sc as plsc

V = 10000
E = 5000
NNZ = 320000
D = 128

NC = 2
NS = 16
NW = NC * NS
PER_W = NNZ // NW
G = 80
NBLK = PER_W // G
EP = 5120
VP = 10240
EP_T = EP // NS
VP_T = VP // NS
YREP = 8

f32 = jnp.float32


def _sc_mesh():
    return plsc.VectorSubcoreMesh(core_axis_name="c", subcore_axis_name="s")


def _hist_body(src_hbm, dst_hbm, zv_hbm, ze_hbm, outv, oute,
               idx_s, idx_d, ones_v, hv_sh, he_sh):
    cid = lax.axis_index("c")
    sid = lax.axis_index("s")
    wid = cid * NS + sid
    pltpu.sync_copy(zv_hbm, hv_sh.at[pl.ds(sid * VP_T, VP_T)])

    @pl.when(sid == 0)
    def _():
        pltpu.sync_copy(ze_hbm, he_sh)

    pltpu.sync_copy(src_hbm.at[wid], idx_s)
    pltpu.sync_copy(dst_hbm.at[wid], idx_d)
    for k in range(G2 // 16):
        ones_v[pl.ds(k * 16, 16)] = jnp.full((16,), 1.0, f32)
    plsc.subcore_barrier()

    @pl.loop(0, NB2)
    def _(g):
        pltpu.sync_copy(ones_v, hv_sh.at[idx_s.at[g]], add=True)
        pltpu.sync_copy(ones_v, he_sh.at[idx_d.at[g]], add=True)

    plsc.subcore_barrier()
    pltpu.sync_copy(hv_sh.at[pl.ds(sid * VP_T, VP_T)],
                    outv.at[pl.ds(cid * VP + sid * VP_T, VP_T)])

    @pl.when(sid == 0)
    def _():
        pltpu.sync_copy(he_sh, oute.at[pl.ds(cid * EP, EP)])


def _hist(src3, dst3, zv, ze):
    k = pl.kernel(
        _hist_body,
        out_type=(jax.ShapeDtypeStruct((NC * VP,), f32),
                  jax.ShapeDtypeStruct((NC * EP,), f32)),
        mesh=_sc_mesh(),
        scratch_types=[
            pltpu.VMEM((NB2, G2), jnp.int32),
            pltpu.VMEM((NB2, G2), jnp.int32),
            pltpu.VMEM((G2,), f32),
            pltpu.VMEM_SHARED((VP,), f32),
            pltpu.VMEM_SHARED((EP,), f32),
        ],
    )
    return k(src3, dst3, zv, ze)


def _v2e_body(xt_hbm, src_hbm, dst_hbm, dv_hbm, zr_hbm, zs_hbm,
              outy, outs,
              idx_s, idx_d, rows_a, rows_b, vals_a, vals_b, y_sh, s_sh,
              sg, sv):
    cid = lax.axis_index("c")
    sid = lax.axis_index("s")
    wid = cid * NS + sid
    pltpu.sync_copy(zr_hbm, y_sh.at[pl.ds(sid * EP_T, EP_T)])

    @pl.when(sid == 0)
    def _():
        pltpu.sync_copy(zs_hbm, s_sh)

    pltpu.sync_copy(src_hbm.at[wid], idx_s)
    pltpu.sync_copy(dst_hbm.at[wid], idx_d)
    pltpu.async_copy(xt_hbm.at[idx_s.at[0]], rows_a, sg)
    pltpu.async_copy(dv_hbm.at[idx_s.at[0]], vals_a, sv)
    plsc.subcore_barrier()

    def step(g, rcur, vcur, rnxt, vnxt, last):
        pltpu.make_async_copy(xt_hbm.at[idx_s.at[g]], rcur, sg).wait()
        pltpu.make_async_copy(dv_hbm.at[idx_s.at[g]], vcur, sv).wait()
        if not last:
            pltpu.async_copy(xt_hbm.at[idx_s.at[g + 1]], rnxt, sg)
            pltpu.async_copy(dv_hbm.at[idx_s.at[g + 1]], vnxt, sv)
        pltpu.sync_copy(rcur, y_sh.at[idx_d.at[g]], add=True)
        pltpu.sync_copy(vcur, s_sh.at[idx_d.at[g]], add=True)

    @pl.loop(0, (NB2 - 2) // 2)
    def _(p):
        step(2 * p, rows_a, vals_a, rows_b, vals_b, False)
        step(2 * p + 1, rows_b, vals_b, rows_a, vals_a, False)

    step(NB2 - 2, rows_a, vals_a, rows_b, vals_b, False)
    step(NB2 - 1, rows_b, vals_b, None, None, True)

    plsc.subcore_barrier()
    pltpu.sync_copy(y_sh.at[pl.ds(sid * EP_T, EP_T)],
                    outy.at[cid, pl.ds(sid * EP_T, EP_T)])

    @pl.when(sid == 0)
    def _():
        pltpu.sync_copy(s_sh, outs.at[pl.ds(cid * EP, EP)])


def _v2e(xt, src3, dst3, dv1, zr, zs):
    k = pl.kernel(
        _v2e_body,
        out_type=(jax.ShapeDtypeStruct((NC, EP, D), f32),
                  jax.ShapeDtypeStruct((NC * EP,), f32)),
        mesh=_sc_mesh(),
        scratch_types=[
            pltpu.VMEM((NB2, G2), jnp.int32),
            pltpu.VMEM((NB2, G2), jnp.int32),
            pltpu.VMEM((G2, D), f32),
            pltpu.VMEM((G2, D), f32),
            pltpu.VMEM((G2,), f32),
            pltpu.VMEM((G2,), f32),
            pltpu.VMEM_SHARED((EP, D), f32),
            pltpu.VMEM_SHARED((EP,), f32),
            pltpu.SemaphoreType.DMA,
            pltpu.SemaphoreType.DMA,
        ],
    )
    return k(xt, src3, dst3, dv1, zr, zs)


YS = 5120
YS_T = YS // NS


G2 = 128
NNZP = NW * VP
NB2 = VP // G2
NCH = 8
CB = 10


def _e2v_body(y_hbm, src_hbm, dst_hbm, zr_hbm, outx,
              idx_s, idx_d, rows_a, rows_b, x_sh, sg):
    cid = lax.axis_index("c")
    sid = lax.axis_index("s")
    wid = cid * NS + sid
    pltpu.sync_copy(zr_hbm, x_sh.at[pl.ds(sid * VP_T, VP_T)])
    plsc.subcore_barrier()

    @pl.loop(0, NCH)
    def _(c):
        pltpu.sync_copy(src_hbm.at[wid, c], idx_s)
        pltpu.sync_copy(dst_hbm.at[wid, c], idx_d)
        pltpu.async_copy(y_hbm.at[idx_d.at[0]], rows_a, sg)

        def step(g, rcur, rnxt, last):
            pltpu.make_async_copy(y_hbm.at[idx_d.at[g]], rcur, sg).wait()
            if not last:
                pltpu.async_copy(y_hbm.at[idx_d.at[g + 1]], rnxt, sg)
            pltpu.sync_copy(rcur, x_sh.at[idx_s.at[g]], add=True)

        @pl.loop(0, (CB - 2) // 2)
        def _(p):
            step(2 * p, rows_a, rows_b, False)
            step(2 * p + 1, rows_b, rows_a, False)

        step(CB - 2, rows_a, rows_b, False)
        step(CB - 1, rows_b, None, True)

    plsc.subcore_barrier()
    pltpu.sync_copy(x_sh.at[pl.ds(sid * VP_T, VP_T)],
                    outx.at[cid, pl.ds(sid * VP_T, VP_T)])


def _e2v(ynorm, src4, dst4, zrv):
    k = pl.kernel(
        _e2v_body,
        out_type=jax.ShapeDtypeStruct((NC, VP, D), f32),
        mesh=_sc_mesh(),
        scratch_types=[
            pltpu.VMEM((CB, G2), jnp.int32),
            pltpu.VMEM((CB, G2), jnp.int32),
            pltpu.VMEM((G2, D), f32),
            pltpu.VMEM((G2, D), f32),
            pltpu.VMEM_SHARED((VP, D), f32),
            pltpu.SemaphoreType.DMA,
        ],
    )
    return k(ynorm, src4, dst4, zrv)


def _matmul_body(x_ref, w_ref, b_ref, o_ref):
    o_ref[...] = lax.dot_general(
        x_ref[...], w_ref[...], (((1,), (1,)), ((), ())),
        preferred_element_type=f32) + b_ref[...]


def _matmul(X, W, b):
    return pl.pallas_call(
        _matmul_body,
        grid=(10,),
        in_specs=[
            pl.BlockSpec((V // 10, D), lambda i: (i, 0)),
            pl.BlockSpec((D, D), lambda i: (0, 0)),
            pl.BlockSpec((1, D), lambda i: (0, 0)),
        ],
        out_specs=pl.BlockSpec((V // 10, D), lambda i: (i, 0)),
        out_shape=jax.ShapeDtypeStruct((V, D), f32),
    )(X, W, b.reshape(1, D))


def _combine_body(d0, d1, c0, c1, dv, dvn, cnt):
    dsum = d0[...] + d1[...]
    dv[...] = dsum
    dvn[...] = jnp.where(dsum > 0, lax.rsqrt(jnp.maximum(dsum, 1e-12)), 0.0)
    cnt[...] = c0[...] + c1[...]


def _combine(d0, d1, c0, c1):
    return pl.pallas_call(
        _combine_body,
        out_shape=(jax.ShapeDtypeStruct((VP // D, D), f32),
                   jax.ShapeDtypeStruct((VP // D, D), f32),
                   jax.ShapeDtypeStruct((EP // D, D), f32)),
    )(d0, d1, c0, c1)


def _norm_body(y0, y1, s0, s1, c, out):
    cc = c[...]
    y = (y0[...] + y1[...]) / jnp.maximum(cc, 1.0)
    de = (s0[...] + s1[...]) / (cc + 1.0)
    fac = jnp.where(cc > 0, lax.rsqrt(jnp.maximum(de, 1e-12)), 1.0)
    out[...] = y * fac


def _norm(y0, y1, s0, s1, cnt):
    nb = 8
    return pl.pallas_call(
        _norm_body,
        grid=(nb * YREP,),
        in_specs=[
            pl.BlockSpec((EP // nb, D), lambda i: (i % nb, 0)),
            pl.BlockSpec((EP // nb, D), lambda i: (i % nb, 0)),
            pl.BlockSpec((EP // nb, 1), lambda i: (i % nb, 0)),
            pl.BlockSpec((EP // nb, 1), lambda i: (i % nb, 0)),
            pl.BlockSpec((EP // nb, 1), lambda i: (i % nb, 0)),
        ],
        out_specs=pl.BlockSpec((EP // nb, D), lambda i: (i, 0)),
        out_shape=jax.ShapeDtypeStruct((YREP * EP, D), f32),
    )(y0, y1, s0, s1, cnt)


def _final_body(x0, x1, dvn, out):
    out[...] = jnp.maximum(dvn[...] * (x0[...] + x1[...]), 0.0)


def _final(x0, x1, dvn):
    nb = 10
    return pl.pallas_call(
        _final_body,
        grid=(nb,),
        in_specs=[
            pl.BlockSpec((V // nb, D), lambda i: (i, 0)),
            pl.BlockSpec((V // nb, D), lambda i: (i, 0)),
            pl.BlockSpec((V // nb, 1), lambda i: (i, 0)),
        ],
        out_specs=pl.BlockSpec((V // nb, D), lambda i: (i, 0)),
        out_shape=jax.ShapeDtypeStruct((V, D), f32),
    )(x0, x1, dvn)


def kernel(X, v2e_src, v2e_dst, W, b):
    npad = NNZP - NNZ
    iota_p = jnp.arange(npad, dtype=jnp.int32)
    srcpd = jnp.concatenate([v2e_src, (iota_p % (VP - V)) + V])
    dstp0 = jnp.concatenate([v2e_dst, (iota_p % (EP - E)) + E])

    xt = _matmul(X, W, b)

    zv = jnp.zeros((VP_T,), f32)
    ze = jnp.zeros((EP,), f32)
    dvp, cep = _hist(srcpd.reshape(NW, NB2, G2), dstp0.reshape(NW, NB2, G2),
                     zv, ze)
    dvp = dvp.reshape(NC, VP)
    cep = cep.reshape(NC, EP)

    dv, dvneg, cnt = _combine(
        dvp[0].reshape(VP // D, D), dvp[1].reshape(VP // D, D),
        cep[0].reshape(EP // D, D), cep[1].reshape(EP // D, D))
    dv1 = dv.reshape(VP)

    srcp2 = jnp.concatenate([v2e_src, iota_p % V]).reshape(NW, NB2, G2)
    dstp2 = jnp.concatenate([v2e_dst, (iota_p % (EP - E)) + E]).reshape(
        NW, NB2, G2)
    zr = jnp.zeros((EP_T, D), f32)
    zs = jnp.zeros((EP,), f32)
    yp, sp = _v2e(xt, srcp2, dstp2, dv1, zr, zs)
    sp = sp.reshape(NC, EP)

    ynorm = _norm(yp[0], yp[1],
                  sp[0].reshape(EP, 1), sp[1].reshape(EP, 1),
                  cnt.reshape(EP, 1))

    dstp = dstp0 + (jnp.arange(NNZP, dtype=jnp.int32) % YREP) * EP
    zrv = jnp.zeros((VP_T, D), f32)
    xp = _e2v(ynorm, srcpd.reshape(NW, NCH, CB, G2),
              dstp.reshape(NW, NCH, CB, G2), zrv)

    return _final(xp[0], xp[1], dvneg.reshape(VP, 1)[:V])


# speedup vs baseline: 1.2006x; 1.1122x over previous
import functools

import jax
import jax.numpy as jnp
from jax import lax
from jax.experimental import pallas as pl
from jax.experimental.pallas import tpu as pltpu
from jax.experimental.pallas import tpu_sc as plsc

V = 10000
E = 5000
NNZ = 320000
D = 128

NC = 2
NS = 16
NW = NC * NS
PER_W = NNZ // NW
G = 80
NBLK = PER_W // G
EP = 5120
VP = 10240
EP_T = EP // NS
VP_T = VP // NS
YREP = 8

f32 = jnp.float32


def _sc_mesh():
    return plsc.VectorSubcoreMesh(core_axis_name="c", subcore_axis_name="s")


def _hist_body(src_hbm, dst_hbm, zv_hbm, ze_hbm, outv, oute,
               idx_s, idx_d, ones_v, hv_sh, he_sh):
    cid = lax.axis_index("c")
    sid = lax.axis_index("s")
    wid = cid * NS + sid
    pltpu.sync_copy(zv_hbm, hv_sh.at[pl.ds(sid * VP_T, VP_T)])

    @pl.when(sid == 0)
    def _():
        pltpu.sync_copy(ze_hbm, he_sh)

    pltpu.sync_copy(src_hbm.at[wid], idx_s)
    pltpu.sync_copy(dst_hbm.at[wid], idx_d)
    for k in range(G2 // 16):
        ones_v[pl.ds(k * 16, 16)] = jnp.full((16,), 1.0, f32)
    plsc.subcore_barrier()

    @pl.loop(0, NB2)
    def _(g):
        pltpu.sync_copy(ones_v, hv_sh.at[idx_s.at[g]], add=True)
        pltpu.sync_copy(ones_v, he_sh.at[idx_d.at[g]], add=True)

    plsc.subcore_barrier()
    pltpu.sync_copy(hv_sh.at[pl.ds(sid * VP_T, VP_T)],
                    outv.at[pl.ds(cid * VP + sid * VP_T, VP_T)])

    @pl.when(sid == 0)
    def _():
        pltpu.sync_copy(he_sh, oute.at[pl.ds(cid * EP, EP)])


def _hist(src3, dst3, zv, ze):
    k = pl.kernel(
        _hist_body,
        out_type=(jax.ShapeDtypeStruct((NC * VP,), f32),
                  jax.ShapeDtypeStruct((NC * EP,), f32)),
        mesh=_sc_mesh(),
        scratch_types=[
            pltpu.VMEM((NB2, G2), jnp.int32),
            pltpu.VMEM((NB2, G2), jnp.int32),
            pltpu.VMEM((G2,), f32),
            pltpu.VMEM_SHARED((VP,), f32),
            pltpu.VMEM_SHARED((EP,), f32),
        ],
    )
    return k(src3, dst3, zv, ze)


def _v2e_body(xt_hbm, src_hbm, dst_hbm, dv_hbm, zr_hbm, zs_hbm,
              outy, outs,
              idx_s, idx_d, rows_a, rows_b, vals_a, vals_b, y_sh, s_sh,
              sg, sv):
    cid = lax.axis_index("c")
    sid = lax.axis_index("s")
    wid = cid * NS + sid
    pltpu.sync_copy(zr_hbm, y_sh.at[pl.ds(sid * EP_T, EP_T)])

    @pl.when(sid == 0)
    def _():
        pltpu.sync_copy(zs_hbm, s_sh)

    pltpu.sync_copy(src_hbm.at[wid], idx_s)
    pltpu.sync_copy(dst_hbm.at[wid], idx_d)
    pltpu.async_copy(xt_hbm.at[idx_s.at[0]], rows_a, sg)
    pltpu.async_copy(dv_hbm.at[idx_s.at[0]], vals_a, sv)
    plsc.subcore_barrier()

    def step(g, rcur, vcur, rnxt, vnxt, last):
        pltpu.make_async_copy(xt_hbm.at[idx_s.at[g]], rcur, sg).wait()
        pltpu.make_async_copy(dv_hbm.at[idx_s.at[g]], vcur, sv).wait()
        if not last:
            pltpu.async_copy(xt_hbm.at[idx_s.at[g + 1]], rnxt, sg)
            pltpu.async_copy(dv_hbm.at[idx_s.at[g + 1]], vnxt, sv)
        pltpu.sync_copy(rcur, y_sh.at[idx_d.at[g]], add=True)
        pltpu.sync_copy(vcur, s_sh.at[idx_d.at[g]], add=True)

    @pl.loop(0, (NB2 - 2) // 2)
    def _(p):
        step(2 * p, rows_a, vals_a, rows_b, vals_b, False)
        step(2 * p + 1, rows_b, vals_b, rows_a, vals_a, False)

    step(NB2 - 2, rows_a, vals_a, rows_b, vals_b, False)
    step(NB2 - 1, rows_b, vals_b, None, None, True)

    plsc.subcore_barrier()
    pltpu.sync_copy(y_sh.at[pl.ds(sid * EP_T, EP_T)],
                    outy.at[cid, pl.ds(sid * EP_T, EP_T)])

    @pl.when(sid == 0)
    def _():
        pltpu.sync_copy(s_sh, outs.at[pl.ds(cid * EP, EP)])


def _v2e(xt, src3, dst3, dv1, zr, zs):
    k = pl.kernel(
        _v2e_body,
        out_type=(jax.ShapeDtypeStruct((NC, EP, D), f32),
                  jax.ShapeDtypeStruct((NC * EP,), f32)),
        mesh=_sc_mesh(),
        scratch_types=[
            pltpu.VMEM((NB2, G2), jnp.int32),
            pltpu.VMEM((NB2, G2), jnp.int32),
            pltpu.VMEM((G2, D), f32),
            pltpu.VMEM((G2, D), f32),
            pltpu.VMEM((G2,), f32),
            pltpu.VMEM((G2,), f32),
            pltpu.VMEM_SHARED((EP, D), f32),
            pltpu.VMEM_SHARED((EP,), f32),
            pltpu.SemaphoreType.DMA,
            pltpu.SemaphoreType.DMA,
        ],
    )
    return k(xt, src3, dst3, dv1, zr, zs)


YS = 5120
YS_T = YS // NS


G2 = 128
NNZP = NW * VP
NB2 = VP // G2
NCH = 8
CB = 10


def _e2v_body(y_hbm, src_hbm, dst_hbm, zr_hbm, outx,
              idx_s, idx_d, rows_a, rows_b, x_sh, sg):
    cid = lax.axis_index("c")
    sid = lax.axis_index("s")
    wid = cid * NS + sid
    pltpu.sync_copy(zr_hbm, x_sh.at[pl.ds(sid * VP_T, VP_T)])
    plsc.subcore_barrier()

    @pl.loop(0, NCH)
    def _(c):
        pltpu.sync_copy(src_hbm.at[wid, c], idx_s)
        pltpu.sync_copy(dst_hbm.at[wid, c], idx_d)
        pltpu.async_copy(y_hbm.at[idx_d.at[0]], rows_a, sg)

        def step(g, rcur, rnxt, last):
            pltpu.make_async_copy(y_hbm.at[idx_d.at[g]], rcur, sg).wait()
            if not last:
                pltpu.async_copy(y_hbm.at[idx_d.at[g + 1]], rnxt, sg)
            pltpu.sync_copy(rcur, x_sh.at[idx_s.at[g]], add=True)

        @pl.loop(0, (CB - 2) // 2)
        def _(p):
            step(2 * p, rows_a, rows_b, False)
            step(2 * p + 1, rows_b, rows_a, False)

        step(CB - 2, rows_a, rows_b, False)
        step(CB - 1, rows_b, None, True)

    plsc.subcore_barrier()
    pltpu.sync_copy(x_sh.at[pl.ds(sid * VP_T, VP_T)],
                    outx.at[cid, pl.ds(sid * VP_T, VP_T)])


def _e2v(ynorm, src4, dst4, zrv):
    k = pl.kernel(
        _e2v_body,
        out_type=jax.ShapeDtypeStruct((NC, VP, D), f32),
        mesh=_sc_mesh(),
        scratch_types=[
            pltpu.VMEM((CB, G2), jnp.int32),
            pltpu.VMEM((CB, G2), jnp.int32),
            pltpu.VMEM((G2, D), f32),
            pltpu.VMEM((G2, D), f32),
            pltpu.VMEM_SHARED((VP, D), f32),
            pltpu.SemaphoreType.DMA,
        ],
    )
    return k(ynorm, src4, dst4, zrv)


def _matmul_body(x_ref, w_ref, b_ref, o_ref):
    o_ref[...] = lax.dot_general(
        x_ref[...], w_ref[...], (((1,), (1,)), ((), ())),
        preferred_element_type=f32) + b_ref[...]


def _matmul(X, W, b):
    return pl.pallas_call(
        _matmul_body,
        grid=(10,),
        in_specs=[
            pl.BlockSpec((V // 10, D), lambda i: (i, 0)),
            pl.BlockSpec((D, D), lambda i: (0, 0)),
            pl.BlockSpec((1, D), lambda i: (0, 0)),
        ],
        out_specs=pl.BlockSpec((V // 10, D), lambda i: (i, 0)),
        out_shape=jax.ShapeDtypeStruct((V, D), f32),
    )(X, W, b.reshape(1, D))


def _combine_body(d0, d1, c0, c1, dv, dvn, cnt):
    dsum = d0[...] + d1[...]
    dv[...] = dsum
    dvn[...] = jnp.where(dsum > 0, lax.rsqrt(jnp.maximum(dsum, 1e-12)), 0.0)
    cnt[...] = c0[...] + c1[...]


def _combine(d0, d1, c0, c1):
    return pl.pallas_call(
        _combine_body,
        out_shape=(jax.ShapeDtypeStruct((VP // D, D), f32),
                   jax.ShapeDtypeStruct((VP // D, D), f32),
                   jax.ShapeDtypeStruct((EP // D, D), f32)),
    )(d0, d1, c0, c1)


def _norm_body(y0, y1, s0, s1, c, out):
    cc = c[...]
    y = (y0[...] + y1[...]) / jnp.maximum(cc, 1.0)
    de = (s0[...] + s1[...]) / (cc + 1.0)
    fac = jnp.where(cc > 0, lax.rsqrt(jnp.maximum(de, 1e-12)), 1.0)
    out[...] = y * fac


def _norm(y0, y1, s0, s1, cnt):
    nb = 8
    return pl.pallas_call(
        _norm_body,
        grid=(nb * YREP,),
        in_specs=[
            pl.BlockSpec((EP // nb, D), lambda i: (i % nb, 0)),
            pl.BlockSpec((EP // nb, D), lambda i: (i % nb, 0)),
            pl.BlockSpec((EP // nb, 1), lambda i: (i % nb, 0)),
            pl.BlockSpec((EP // nb, 1), lambda i: (i % nb, 0)),
            pl.BlockSpec((EP // nb, 1), lambda i: (i % nb, 0)),
        ],
        out_specs=pl.BlockSpec((EP // nb, D), lambda i: (i, 0)),
        out_shape=jax.ShapeDtypeStruct((YREP * EP, D), f32),
    )(y0, y1, s0, s1, cnt)


def _final_body(x0, x1, dvn, out):
    out[...] = jnp.maximum(dvn[...] * (x0[...] + x1[...]), 0.0)


def _final(x0, x1, dvn):
    nb = 10
    return pl.pallas_call(
        _final_body,
        grid=(nb,),
        in_specs=[
            pl.BlockSpec((V // nb, D), lambda i: (i, 0)),
            pl.BlockSpec((V // nb, D), lambda i: (i, 0)),
            pl.BlockSpec((V // nb, 1), lambda i: (i, 0)),
        ],
        out_specs=pl.BlockSpec((V // nb, D), lambda i: (i, 0)),
        out_shape=jax.ShapeDtypeStruct((V, D), f32),
    )(x0, x1, dvn)


def kernel(X, v2e_src, v2e_dst, W, b):
    npad = NNZP - NNZ
    iota_p = jnp.arange(npad, dtype=jnp.int32)
    srcpd = jnp.concatenate([v2e_src, (iota_p % (VP - V)) + V])
    dstp0 = jnp.concatenate([v2e_dst, (iota_p % (EP - E)) + E])

    xt = _matmul(X, W, b)

    zv = jnp.zeros((VP_T,), f32)
    ze = jnp.zeros((EP,), f32)
    dvp, cep = _hist(srcpd.reshape(NW, NB2, G2), dstp0.reshape(NW, NB2, G2),
                     zv, ze)
    dvp = dvp.reshape(NC, VP)
    cep = cep.reshape(NC, EP)

    dv, dvneg, cnt = _combine(
        dvp[0].reshape(VP // D, D), dvp[1].reshape(VP // D, D),
        cep[0].reshape(EP // D, D), cep[1].reshape(EP // D, D))
    dv1 = dv.reshape(VP)

    srcp2 = jnp.concatenate([v2e_src, iota_p % V]).reshape(NW, NB2, G2)
    dstp2 = jnp.concatenate([v2e_dst, (iota_p % (EP - E)) + E]).reshape(
        NW, NB2, G2)
    zr = jnp.zeros((EP_T, D), f32)
    zs = jnp.zeros((EP,), f32)
    yp, sp = _v2e(xt, srcp2, dstp2, dv1, zr, zs)
    sp = sp.reshape(NC, EP)

    ynorm = _norm(yp[0], yp[1],
                  sp[0].reshape(EP, 1), sp[1].reshape(EP, 1),
                  cnt.reshape(EP, 1))

    def _wtr(a):
        return a.reshape(NW, NB2, G2).transpose(0, 2, 1).reshape(-1)

    dstp = _wtr(dstp0) + (jnp.arange(NNZP, dtype=jnp.int32) % YREP) * EP
    zrv = jnp.zeros((VP_T, D), f32)
    xp = _e2v(ynorm, _wtr(srcpd).reshape(NW, NCH, CB, G2),
              dstp.reshape(NW, NCH, CB, G2), zrv)

    return _final(xp[0], xp[1], dvneg.reshape(VP, 1)[:V])


# speedup vs baseline: 1.3968x; 1.1634x over previous
import functools

import jax
import jax.numpy as jnp
from jax import lax
from jax.experimental import pallas as pl
from jax.experimental.pallas import tpu as pltpu
from jax.experimental.pallas import tpu_sc as plsc

V = 10000
E = 5000
NNZ = 320000
D = 128

NC = 2
NS = 16
NW = NC * NS
PER_W = NNZ // NW
G = 80
NBLK = PER_W // G
EP = 5120
VP = 10240
EP_T = EP // NS
VP_T = VP // NS
YREP = 1

f32 = jnp.float32


def _sc_mesh():
    return plsc.VectorSubcoreMesh(core_axis_name="c", subcore_axis_name="s")


def _hist_body(src_hbm, dst_hbm, zv_hbm, ze_hbm, outv, oute,
               idx_s, idx_d, ones_v, hv_sh, he_sh):
    cid = lax.axis_index("c")
    sid = lax.axis_index("s")
    wid = cid * NS + sid
    pltpu.sync_copy(zv_hbm, hv_sh.at[pl.ds(sid * VP_T, VP_T)])

    @pl.when(sid == 0)
    def _():
        pltpu.sync_copy(ze_hbm, he_sh)

    pltpu.sync_copy(src_hbm.at[wid], idx_s)
    pltpu.sync_copy(dst_hbm.at[wid], idx_d)
    for k in range(G2 // 16):
        ones_v[pl.ds(k * 16, 16)] = jnp.full((16,), 1.0, f32)
    plsc.subcore_barrier()

    @pl.loop(0, NB2)
    def _(g):
        pltpu.sync_copy(ones_v, hv_sh.at[idx_s.at[g]], add=True)
        pltpu.sync_copy(ones_v, he_sh.at[idx_d.at[g]], add=True)

    plsc.subcore_barrier()
    pltpu.sync_copy(hv_sh.at[pl.ds(sid * VP_T, VP_T)],
                    outv.at[pl.ds(cid * VP + sid * VP_T, VP_T)])

    @pl.when(sid == 0)
    def _():
        pltpu.sync_copy(he_sh, oute.at[pl.ds(cid * EP, EP)])


def _hist(src3, dst3, zv, ze):
    k = pl.kernel(
        _hist_body,
        out_type=(jax.ShapeDtypeStruct((NC * VP,), f32),
                  jax.ShapeDtypeStruct((NC * EP,), f32)),
        mesh=_sc_mesh(),
        scratch_types=[
            pltpu.VMEM((NB2, G2), jnp.int32),
            pltpu.VMEM((NB2, G2), jnp.int32),
            pltpu.VMEM((G2,), f32),
            pltpu.VMEM_SHARED((VP,), f32),
            pltpu.VMEM_SHARED((EP,), f32),
        ],
    )
    return k(src3, dst3, zv, ze)


def _v2e_body(xt_hbm, src_hbm, dst_hbm, dv_hbm, zr_hbm, zs_hbm,
              outy, outs,
              idx_s, idx_d, rows_a, rows_b, vals_a, vals_b, y_sh, s_sh,
              sg, sv):
    cid = lax.axis_index("c")
    sid = lax.axis_index("s")
    wid = cid * NS + sid
    pltpu.sync_copy(zr_hbm, y_sh.at[pl.ds(sid * EP_T, EP_T)])

    @pl.when(sid == 0)
    def _():
        pltpu.sync_copy(zs_hbm, s_sh)

    pltpu.sync_copy(src_hbm.at[wid], idx_s)
    pltpu.sync_copy(dst_hbm.at[wid], idx_d)
    pltpu.async_copy(xt_hbm.at[idx_s.at[0]], rows_a, sg)
    pltpu.async_copy(dv_hbm.at[idx_s.at[0]], vals_a, sv)
    plsc.subcore_barrier()

    def step(g, rcur, vcur, rnxt, vnxt, last):
        pltpu.make_async_copy(xt_hbm.at[idx_s.at[g]], rcur, sg).wait()
        pltpu.make_async_copy(dv_hbm.at[idx_s.at[g]], vcur, sv).wait()
        if not last:
            pltpu.async_copy(xt_hbm.at[idx_s.at[g + 1]], rnxt, sg)
            pltpu.async_copy(dv_hbm.at[idx_s.at[g + 1]], vnxt, sv)
        pltpu.sync_copy(rcur, y_sh.at[idx_d.at[g]], add=True)
        pltpu.sync_copy(vcur, s_sh.at[idx_d.at[g]], add=True)

    @pl.loop(0, (NB2 - 2) // 2)
    def _(p):
        step(2 * p, rows_a, vals_a, rows_b, vals_b, False)
        step(2 * p + 1, rows_b, vals_b, rows_a, vals_a, False)

    step(NB2 - 2, rows_a, vals_a, rows_b, vals_b, False)
    step(NB2 - 1, rows_b, vals_b, None, None, True)

    plsc.subcore_barrier()
    pltpu.sync_copy(y_sh.at[pl.ds(sid * EP_T, EP_T)],
                    outy.at[cid, pl.ds(sid * EP_T, EP_T)])

    @pl.when(sid == 0)
    def _():
        pltpu.sync_copy(s_sh, outs.at[pl.ds(cid * EP, EP)])


def _v2e(xt, src3, dst3, dv1, zr, zs):
    k = pl.kernel(
        _v2e_body,
        out_type=(jax.ShapeDtypeStruct((NC, EP, D), f32),
                  jax.ShapeDtypeStruct((NC * EP,), f32)),
        mesh=_sc_mesh(),
        scratch_types=[
            pltpu.VMEM((NB2, G2), jnp.int32),
            pltpu.VMEM((NB2, G2), jnp.int32),
            pltpu.VMEM((G2, D), f32),
            pltpu.VMEM((G2, D), f32),
            pltpu.VMEM((G2,), f32),
            pltpu.VMEM((G2,), f32),
            pltpu.VMEM_SHARED((EP, D), f32),
            pltpu.VMEM_SHARED((EP,), f32),
            pltpu.SemaphoreType.DMA,
            pltpu.SemaphoreType.DMA,
        ],
    )
    return k(xt, src3, dst3, dv1, zr, zs)


YS = 5120
YS_T = YS // NS


G2 = 128
NNZP = NW * VP
NB2 = VP // G2
NCH = 8
CB = 10


def _e2v_body(y_hbm, src_hbm, dst_hbm, zr_hbm, outx,
              idx_s, idx_d, rows_a, rows_b, x_sh, sg):
    cid = lax.axis_index("c")
    sid = lax.axis_index("s")
    wid = cid * NS + sid
    pltpu.sync_copy(zr_hbm, x_sh.at[pl.ds(sid * VP_T, VP_T)])
    plsc.subcore_barrier()

    @pl.loop(0, NCH)
    def _(c):
        pltpu.sync_copy(src_hbm.at[wid, c], idx_s)
        pltpu.sync_copy(dst_hbm.at[wid, c], idx_d)
        pltpu.async_copy(y_hbm.at[idx_d.at[0]], rows_a, sg)

        def step(g, rcur, rnxt, last):
            pltpu.make_async_copy(y_hbm.at[idx_d.at[g]], rcur, sg).wait()
            if not last:
                pltpu.async_copy(y_hbm.at[idx_d.at[g + 1]], rnxt, sg)
            pltpu.sync_copy(rcur, x_sh.at[idx_s.at[g]], add=True)

        @pl.loop(0, (CB - 2) // 2)
        def _(p):
            step(2 * p, rows_a, rows_b, False)
            step(2 * p + 1, rows_b, rows_a, False)

        step(CB - 2, rows_a, rows_b, False)
        step(CB - 1, rows_b, None, True)

    plsc.subcore_barrier()
    pltpu.sync_copy(x_sh.at[pl.ds(sid * VP_T, VP_T)],
                    outx.at[cid, pl.ds(sid * VP_T, VP_T)])


def _e2v(ynorm, src4, dst4, zrv):
    k = pl.kernel(
        _e2v_body,
        out_type=jax.ShapeDtypeStruct((NC, VP, D), f32),
        mesh=_sc_mesh(),
        scratch_types=[
            pltpu.VMEM((CB, G2), jnp.int32),
            pltpu.VMEM((CB, G2), jnp.int32),
            pltpu.VMEM((G2, D), f32),
            pltpu.VMEM((G2, D), f32),
            pltpu.VMEM_SHARED((VP, D), f32),
            pltpu.SemaphoreType.DMA,
        ],
    )
    return k(ynorm, src4, dst4, zrv)


def _matmul_body(x_ref, w_ref, b_ref, o_ref):
    o_ref[...] = lax.dot_general(
        x_ref[...], w_ref[...], (((1,), (1,)), ((), ())),
        preferred_element_type=f32) + b_ref[...]


def _matmul(X, W, b):
    return pl.pallas_call(
        _matmul_body,
        grid=(10,),
        in_specs=[
            pl.BlockSpec((V // 10, D), lambda i: (i, 0)),
            pl.BlockSpec((D, D), lambda i: (0, 0)),
            pl.BlockSpec((1, D), lambda i: (0, 0)),
        ],
        out_specs=pl.BlockSpec((V // 10, D), lambda i: (i, 0)),
        out_shape=jax.ShapeDtypeStruct((V, D), f32),
    )(X, W, b.reshape(1, D))


def _combine_body(d0, d1, c0, c1, dv, dvn, cnt):
    dsum = d0[...] + d1[...]
    dv[...] = dsum
    dvn[...] = jnp.where(dsum > 0, lax.rsqrt(jnp.maximum(dsum, 1e-12)), 0.0)
    cnt[...] = c0[...] + c1[...]


def _combine(d0, d1, c0, c1):
    return pl.pallas_call(
        _combine_body,
        out_shape=(jax.ShapeDtypeStruct((VP // D, D), f32),
                   jax.ShapeDtypeStruct((VP // D, D), f32),
                   jax.ShapeDtypeStruct((EP // D, D), f32)),
    )(d0, d1, c0, c1)


def _norm_body(y0, y1, s0, s1, c, out):
    cc = c[...]
    y = (y0[...] + y1[...]) / jnp.maximum(cc, 1.0)
    de = (s0[...] + s1[...]) / (cc + 1.0)
    fac = jnp.where(cc > 0, lax.rsqrt(jnp.maximum(de, 1e-12)), 1.0)
    out[...] = y * fac


def _norm(y0, y1, s0, s1, cnt):
    nb = 8
    return pl.pallas_call(
        _norm_body,
        grid=(nb * YREP,),
        in_specs=[
            pl.BlockSpec((EP // nb, D), lambda i: (i % nb, 0)),
            pl.BlockSpec((EP // nb, D), lambda i: (i % nb, 0)),
            pl.BlockSpec((EP // nb, 1), lambda i: (i % nb, 0)),
            pl.BlockSpec((EP // nb, 1), lambda i: (i % nb, 0)),
            pl.BlockSpec((EP // nb, 1), lambda i: (i % nb, 0)),
        ],
        out_specs=pl.BlockSpec((EP // nb, D), lambda i: (i, 0)),
        out_shape=jax.ShapeDtypeStruct((YREP * EP, D), f32),
    )(y0, y1, s0, s1, cnt)


def _final_body(x0, x1, dvn, out):
    out[...] = jnp.maximum(dvn[...] * (x0[...] + x1[...]), 0.0)


def _final(x0, x1, dvn):
    nb = 10
    return pl.pallas_call(
        _final_body,
        grid=(nb,),
        in_specs=[
            pl.BlockSpec((V // nb, D), lambda i: (i, 0)),
            pl.BlockSpec((V // nb, D), lambda i: (i, 0)),
            pl.BlockSpec((V // nb, 1), lambda i: (i, 0)),
        ],
        out_specs=pl.BlockSpec((V // nb, D), lambda i: (i, 0)),
        out_shape=jax.ShapeDtypeStruct((V, D), f32),
    )(x0, x1, dvn)


def kernel(X, v2e_src, v2e_dst, W, b):
    npad = NNZP - NNZ
    iota_p = jnp.arange(npad, dtype=jnp.int32)
    srcpd = jnp.concatenate([v2e_src, (iota_p % (VP - V)) + V])
    dstp0 = jnp.concatenate([v2e_dst, (iota_p % (EP - E)) + E])

    xt = _matmul(X, W, b)

    zv = jnp.zeros((VP_T,), f32)
    ze = jnp.zeros((EP,), f32)
    dvp, cep = _hist(srcpd.reshape(NW, NB2, G2), dstp0.reshape(NW, NB2, G2),
                     zv, ze)
    dvp = dvp.reshape(NC, VP)
    cep = cep.reshape(NC, EP)

    dv, dvneg, cnt = _combine(
        dvp[0].reshape(VP // D, D), dvp[1].reshape(VP // D, D),
        cep[0].reshape(EP // D, D), cep[1].reshape(EP // D, D))
    dv1 = dv.reshape(VP)

    srcp2 = jnp.concatenate([v2e_src, iota_p % V]).reshape(NW, NB2, G2)
    dstp2 = jnp.concatenate([v2e_dst, (iota_p % (EP - E)) + E]).reshape(
        NW, NB2, G2)
    zr = jnp.zeros((EP_T, D), f32)
    zs = jnp.zeros((EP,), f32)
    yp, sp = _v2e(xt, srcp2, dstp2, dv1, zr, zs)
    sp = sp.reshape(NC, EP)

    ynorm = _norm(yp[0], yp[1],
                  sp[0].reshape(EP, 1), sp[1].reshape(EP, 1),
                  cnt.reshape(EP, 1))

    def _wtr(a):
        return a.reshape(NW, NB2, G2).transpose(0, 2, 1).reshape(-1)

    dstp = _wtr(dstp0) + (jnp.arange(NNZP, dtype=jnp.int32) % YREP) * EP
    zrv = jnp.zeros((VP_T, D), f32)
    xp = _e2v(ynorm, _wtr(srcpd).reshape(NW, NCH, CB, G2),
              dstp.reshape(NW, NCH, CB, G2), zrv)

    return _final(xp[0], xp[1], dvneg.reshape(VP, 1)[:V])


# speedup vs baseline: 1.4026x; 1.0042x over previous
import functools

import jax
import jax.numpy as jnp
from jax import lax
from jax.experimental import pallas as pl
from jax.experimental.pallas import tpu as pltpu
from jax.experimental.pallas import tpu_sc as plsc

V = 10000
E = 5000
NNZ = 320000
D = 128

NC = 2
NS = 16
NW = NC * NS
PER_W = NNZ // NW
G = 80
NBLK = PER_W // G
EP = 5120
VP = 10240
EP_T = EP // NS
VP_T = VP // NS
YREP = 1

f32 = jnp.float32


def _sc_mesh():
    return plsc.VectorSubcoreMesh(core_axis_name="c", subcore_axis_name="s")


def _hist_body(src_hbm, dst_hbm, zv_hbm, ze_hbm, outv, oute,
               idx_s, idx_d, ones_v, hv_sh, he_sh):
    cid = lax.axis_index("c")
    sid = lax.axis_index("s")
    wid = cid * NS + sid
    pltpu.sync_copy(zv_hbm, hv_sh.at[pl.ds(sid * VP_T, VP_T)])

    @pl.when(sid == 0)
    def _():
        pltpu.sync_copy(ze_hbm, he_sh)

    pltpu.sync_copy(src_hbm.at[wid], idx_s)
    pltpu.sync_copy(dst_hbm.at[wid], idx_d)
    for k in range(G2 // 16):
        ones_v[pl.ds(k * 16, 16)] = jnp.full((16,), 1.0, f32)
    plsc.subcore_barrier()

    @pl.loop(0, NB2)
    def _(g):
        pltpu.sync_copy(ones_v, hv_sh.at[idx_s.at[g]], add=True)
        pltpu.sync_copy(ones_v, he_sh.at[idx_d.at[g]], add=True)

    plsc.subcore_barrier()
    pltpu.sync_copy(hv_sh.at[pl.ds(sid * VP_T, VP_T)],
                    outv.at[pl.ds(cid * VP + sid * VP_T, VP_T)])

    @pl.when(sid == 0)
    def _():
        pltpu.sync_copy(he_sh, oute.at[pl.ds(cid * EP, EP)])


def _hist(src3, dst3, zv, ze):
    k = pl.kernel(
        _hist_body,
        out_type=(jax.ShapeDtypeStruct((NC * VP,), f32),
                  jax.ShapeDtypeStruct((NC * EP,), f32)),
        mesh=_sc_mesh(),
        scratch_types=[
            pltpu.VMEM((NB2, G2), jnp.int32),
            pltpu.VMEM((NB2, G2), jnp.int32),
            pltpu.VMEM((G2,), f32),
            pltpu.VMEM_SHARED((VP,), f32),
            pltpu.VMEM_SHARED((EP,), f32),
        ],
    )
    return k(src3, dst3, zv, ze)


def _v2e_body(xt_hbm, src_hbm, dst_hbm, dv_hbm, zr_hbm, zs_hbm,
              outy, outs,
              idx_s, idx_d, rows_a, rows_b, vals_a, vals_b, y_sh, s_sh,
              sg, sv):
    cid = lax.axis_index("c")
    sid = lax.axis_index("s")
    wid = cid * NS + sid
    pltpu.sync_copy(zr_hbm, y_sh.at[pl.ds(sid * EP_T, EP_T)])

    @pl.when(sid == 0)
    def _():
        pltpu.sync_copy(zs_hbm, s_sh)

    pltpu.sync_copy(src_hbm.at[wid], idx_s)
    pltpu.sync_copy(dst_hbm.at[wid], idx_d)
    pltpu.async_copy(xt_hbm.at[idx_s.at[0]], rows_a, sg)
    pltpu.async_copy(dv_hbm.at[idx_s.at[0]], vals_a, sv)
    plsc.subcore_barrier()

    def step(g, rcur, vcur, rnxt, vnxt, last):
        pltpu.make_async_copy(xt_hbm.at[idx_s.at[g]], rcur, sg).wait()
        pltpu.make_async_copy(dv_hbm.at[idx_s.at[g]], vcur, sv).wait()
        if not last:
            pltpu.async_copy(xt_hbm.at[idx_s.at[g + 1]], rnxt, sg)
            pltpu.async_copy(dv_hbm.at[idx_s.at[g + 1]], vnxt, sv)
        pltpu.sync_copy(rcur, y_sh.at[idx_d.at[g]], add=True)
        pltpu.sync_copy(vcur, s_sh.at[idx_d.at[g]], add=True)

    @pl.loop(0, (NB2 - 2) // 2)
    def _(p):
        step(2 * p, rows_a, vals_a, rows_b, vals_b, False)
        step(2 * p + 1, rows_b, vals_b, rows_a, vals_a, False)

    step(NB2 - 2, rows_a, vals_a, rows_b, vals_b, False)
    step(NB2 - 1, rows_b, vals_b, None, None, True)

    plsc.subcore_barrier()
    pltpu.sync_copy(y_sh.at[pl.ds(sid * EP_T, EP_T)],
                    outy.at[cid, pl.ds(sid * EP_T, EP_T)])

    @pl.when(sid == 0)
    def _():
        pltpu.sync_copy(s_sh, outs.at[pl.ds(cid * EP, EP)])


def _v2e(xt, src3, dst3, dv1, zr, zs):
    k = pl.kernel(
        _v2e_body,
        out_type=(jax.ShapeDtypeStruct((NC, EP, D), f32),
                  jax.ShapeDtypeStruct((NC * EP,), f32)),
        mesh=_sc_mesh(),
        scratch_types=[
            pltpu.VMEM((NB2, G2), jnp.int32),
            pltpu.VMEM((NB2, G2), jnp.int32),
            pltpu.VMEM((G2, D), f32),
            pltpu.VMEM((G2, D), f32),
            pltpu.VMEM((G2,), f32),
            pltpu.VMEM((G2,), f32),
            pltpu.VMEM_SHARED((EP, D), f32),
            pltpu.VMEM_SHARED((EP,), f32),
            pltpu.SemaphoreType.DMA,
            pltpu.SemaphoreType.DMA,
        ],
    )
    return k(xt, src3, dst3, dv1, zr, zs)


YS = 5120
YS_T = YS // NS


G2 = 128
NNZP = NW * VP
NB2 = VP // G2
NCH = 8
CB = 10


def _e2v_body(y_hbm, src_hbm, dst_hbm, zr_hbm, outx,
              idx_s, idx_d, rows_a, rows_b, x_sh, sg):
    cid = lax.axis_index("c")
    sid = lax.axis_index("s")
    wid = cid * NS + sid
    pltpu.sync_copy(zr_hbm, x_sh.at[pl.ds(sid * VP_T, VP_T)])
    plsc.subcore_barrier()

    @pl.loop(0, NCH)
    def _(c):
        pltpu.sync_copy(src_hbm.at[wid, c], idx_s)
        pltpu.sync_copy(dst_hbm.at[wid, c], idx_d)
        pltpu.async_copy(y_hbm.at[idx_d.at[0]], rows_a, sg)

        def step(g, rcur, rnxt, last):
            pltpu.make_async_copy(y_hbm.at[idx_d.at[g]], rcur, sg).wait()
            if not last:
                pltpu.async_copy(y_hbm.at[idx_d.at[g + 1]], rnxt, sg)
            pltpu.sync_copy(rcur, x_sh.at[idx_s.at[g]], add=True)

        @pl.loop(0, (CB - 2) // 2)
        def _(p):
            step(2 * p, rows_a, rows_b, False)
            step(2 * p + 1, rows_b, rows_a, False)

        step(CB - 2, rows_a, rows_b, False)
        step(CB - 1, rows_b, None, True)

    plsc.subcore_barrier()
    pltpu.sync_copy(x_sh.at[pl.ds(sid * VP_T, VP_T)],
                    outx.at[cid, pl.ds(sid * VP_T, VP_T)])


def _e2v(ynorm, src4, dst4, zrv):
    k = pl.kernel(
        _e2v_body,
        out_type=jax.ShapeDtypeStruct((NC, VP, D), f32),
        mesh=_sc_mesh(),
        scratch_types=[
            pltpu.VMEM((CB, G2), jnp.int32),
            pltpu.VMEM((CB, G2), jnp.int32),
            pltpu.VMEM((G2, D), f32),
            pltpu.VMEM((G2, D), f32),
            pltpu.VMEM_SHARED((VP, D), f32),
            pltpu.SemaphoreType.DMA,
        ],
    )
    return k(ynorm, src4, dst4, zrv)


def _matmul_body(x_ref, w_ref, b_ref, o_ref):
    o_ref[...] = lax.dot_general(
        x_ref[...], w_ref[...], (((1,), (1,)), ((), ())),
        preferred_element_type=f32) + b_ref[...]


def _matmul(X, W, b):
    return pl.pallas_call(
        _matmul_body,
        grid=(10,),
        in_specs=[
            pl.BlockSpec((V // 10, D), lambda i: (i, 0)),
            pl.BlockSpec((D, D), lambda i: (0, 0)),
            pl.BlockSpec((1, D), lambda i: (0, 0)),
        ],
        out_specs=pl.BlockSpec((V // 10, D), lambda i: (i, 0)),
        out_shape=jax.ShapeDtypeStruct((V, D), f32),
    )(X, W, b.reshape(1, D))


def _combine_body(d0, d1, c0, c1, dv, dvn, cnt):
    dsum = d0[...] + d1[...]
    dv[...] = dsum
    dvn[...] = jnp.where(dsum > 0, lax.rsqrt(jnp.maximum(dsum, 1e-12)), 0.0)
    cnt[...] = c0[...] + c1[...]


def _combine(d0, d1, c0, c1):
    return pl.pallas_call(
        _combine_body,
        out_shape=(jax.ShapeDtypeStruct((VP // D, D), f32),
                   jax.ShapeDtypeStruct((VP // D, D), f32),
                   jax.ShapeDtypeStruct((EP // D, D), f32)),
    )(d0, d1, c0, c1)


def _norm_body(y0, y1, s0, s1, c, out):
    cc = c[...]
    y = (y0[...] + y1[...]) / jnp.maximum(cc, 1.0)
    de = (s0[...] + s1[...]) / (cc + 1.0)
    fac = jnp.where(cc > 0, lax.rsqrt(jnp.maximum(de, 1e-12)), 1.0)
    out[...] = y * fac


def _norm(y0, y1, s0, s1, cnt):
    nb = 8
    return pl.pallas_call(
        _norm_body,
        grid=(nb * YREP,),
        in_specs=[
            pl.BlockSpec((EP // nb, D), lambda i: (i % nb, 0)),
            pl.BlockSpec((EP // nb, D), lambda i: (i % nb, 0)),
            pl.BlockSpec((EP // nb, 1), lambda i: (i % nb, 0)),
            pl.BlockSpec((EP // nb, 1), lambda i: (i % nb, 0)),
            pl.BlockSpec((EP // nb, 1), lambda i: (i % nb, 0)),
        ],
        out_specs=pl.BlockSpec((EP // nb, D), lambda i: (i, 0)),
        out_shape=jax.ShapeDtypeStruct((YREP * EP, D), f32),
    )(y0, y1, s0, s1, cnt)


def _final_body(x0, x1, dvn, out):
    out[...] = jnp.maximum(dvn[...] * (x0[...] + x1[...]), 0.0)


def _final(x0, x1, dvn):
    nb = 10
    return pl.pallas_call(
        _final_body,
        grid=(nb,),
        in_specs=[
            pl.BlockSpec((V // nb, D), lambda i: (i, 0)),
            pl.BlockSpec((V // nb, D), lambda i: (i, 0)),
            pl.BlockSpec((V // nb, 1), lambda i: (i, 0)),
        ],
        out_specs=pl.BlockSpec((V // nb, D), lambda i: (i, 0)),
        out_shape=jax.ShapeDtypeStruct((V, D), f32),
    )(x0, x1, dvn)


def kernel(X, v2e_src, v2e_dst, W, b):
    npad = NNZP - NNZ
    iota_p = jnp.arange(npad, dtype=jnp.int32)
    srcpd = jnp.concatenate([v2e_src, (iota_p % (VP - V)) + V])
    dstp0 = jnp.concatenate([v2e_dst, (iota_p % (EP - E)) + E])

    xt = _matmul(X, W, b)

    zv = jnp.zeros((VP_T,), f32)
    ze = jnp.zeros((EP,), f32)
    dvp, cep = _hist(srcpd.reshape(NW, NB2, G2), dstp0.reshape(NW, NB2, G2),
                     zv, ze)
    dvp = dvp.reshape(NC, VP)
    cep = cep.reshape(NC, EP)

    dv, dvneg, cnt = _combine(
        dvp[0].reshape(VP // D, D), dvp[1].reshape(VP // D, D),
        cep[0].reshape(EP // D, D), cep[1].reshape(EP // D, D))
    dv1 = dv.reshape(VP)

    def _wtr2(a):
        return a.reshape(NW, NB2, G2).transpose(0, 2, 1).reshape(NW, NB2, G2)

    srcp2 = _wtr2(jnp.concatenate([v2e_src, iota_p % V]))
    dstp2 = _wtr2(jnp.concatenate([v2e_dst, (iota_p % (EP - E)) + E]))
    zr = jnp.zeros((EP_T, D), f32)
    zs = jnp.zeros((EP,), f32)
    yp, sp = _v2e(xt, srcp2, dstp2, dv1, zr, zs)
    sp = sp.reshape(NC, EP)

    ynorm = _norm(yp[0], yp[1],
                  sp[0].reshape(EP, 1), sp[1].reshape(EP, 1),
                  cnt.reshape(EP, 1))

    def _wtr(a):
        return a.reshape(NW, NB2, G2).transpose(0, 2, 1).reshape(-1)

    dstp = _wtr(dstp0) + (jnp.arange(NNZP, dtype=jnp.int32) % YREP) * EP
    zrv = jnp.zeros((VP_T, D), f32)
    xp = _e2v(ynorm, _wtr(srcpd).reshape(NW, NCH, CB, G2),
              dstp.reshape(NW, NCH, CB, G2), zrv)

    return _final(xp[0], xp[1], dvneg.reshape(VP, 1)[:V])


# speedup vs baseline: 1.4062x; 1.0025x over previous
import jax
import jax.numpy as jnp
from jax import lax
from jax.experimental import pallas as pl
from jax.experimental.pallas import tpu as pltpu
from jax.experimental.pallas import tpu_sc as plsc

V = 10000
E = 5000
NNZ = 320000
D = 128

NC = 2
NS = 16
NW = NC * NS
EP = 5120
VP = 10240
EP_T = EP // NS
VP_T = VP // NS
YREP = 1

f32 = jnp.float32


def _sc_mesh():
    return plsc.VectorSubcoreMesh(core_axis_name="c", subcore_axis_name="s")


def _hist_body(src_hbm, dst_hbm, zv_hbm, ze_hbm, outv, oute,
               idx_s, idx_d, ones_v, hv_sh, he_sh):
    cid = lax.axis_index("c")
    sid = lax.axis_index("s")
    wid = cid * NS + sid
    pltpu.sync_copy(zv_hbm, hv_sh.at[pl.ds(sid * VP_T, VP_T)])

    @pl.when(sid == 0)
    def _():
        pltpu.sync_copy(ze_hbm, he_sh)

    pltpu.sync_copy(src_hbm.at[wid], idx_s)
    pltpu.sync_copy(dst_hbm.at[wid], idx_d)
    for k in range(G2 // 16):
        ones_v[pl.ds(k * 16, 16)] = jnp.full((16,), 1.0, f32)
    plsc.subcore_barrier()

    @pl.loop(0, NB2)
    def _(g):
        pltpu.sync_copy(ones_v, hv_sh.at[idx_s.at[g]], add=True)
        pltpu.sync_copy(ones_v, he_sh.at[idx_d.at[g]], add=True)

    plsc.subcore_barrier()
    pltpu.sync_copy(hv_sh.at[pl.ds(sid * VP_T, VP_T)],
                    outv.at[pl.ds(cid * VP + sid * VP_T, VP_T)])

    @pl.when(sid == 0)
    def _():
        pltpu.sync_copy(he_sh, oute.at[pl.ds(cid * EP, EP)])


def _hist(src3, dst3, zv, ze):
    k = pl.kernel(
        _hist_body,
        out_type=(jax.ShapeDtypeStruct((NC * VP,), f32),
                  jax.ShapeDtypeStruct((NC * EP,), f32)),
        mesh=_sc_mesh(),
        scratch_types=[
            pltpu.VMEM((NB2, G2), jnp.int32),
            pltpu.VMEM((NB2, G2), jnp.int32),
            pltpu.VMEM((G2,), f32),
            pltpu.VMEM_SHARED((VP,), f32),
            pltpu.VMEM_SHARED((EP,), f32),
        ],
    )
    return k(src3, dst3, zv, ze)


def _v2e_body(xt_hbm, src_hbm, dst_hbm, dv_hbm, zr_hbm, zs_hbm,
              outy, outs,
              idx_s, idx_d, rows_a, rows_b, vals_a, vals_b, y_sh, s_sh,
              sg, sv):
    cid = lax.axis_index("c")
    sid = lax.axis_index("s")
    wid = cid * NS + sid
    pltpu.sync_copy(zr_hbm, y_sh.at[pl.ds(sid * EP_T, EP_T)])

    @pl.when(sid == 0)
    def _():
        pltpu.sync_copy(zs_hbm, s_sh)

    pltpu.sync_copy(src_hbm.at[wid], idx_s)
    pltpu.sync_copy(dst_hbm.at[wid], idx_d)
    pltpu.async_copy(xt_hbm.at[idx_s.at[0]], rows_a, sg)
    pltpu.async_copy(dv_hbm.at[idx_s.at[0]], vals_a, sv)
    plsc.subcore_barrier()

    def step(g, rcur, vcur, rnxt, vnxt, last):
        pltpu.make_async_copy(xt_hbm.at[idx_s.at[g]], rcur, sg).wait()
        pltpu.make_async_copy(dv_hbm.at[idx_s.at[g]], vcur, sv).wait()
        if not last:
            pltpu.async_copy(xt_hbm.at[idx_s.at[g + 1]], rnxt, sg)
            pltpu.async_copy(dv_hbm.at[idx_s.at[g + 1]], vnxt, sv)
        pltpu.sync_copy(rcur, y_sh.at[idx_d.at[g]], add=True)
        pltpu.sync_copy(vcur, s_sh.at[idx_d.at[g]], add=True)

    @pl.loop(0, (NB2 - 2) // 2)
    def _(p):
        step(2 * p, rows_a, vals_a, rows_b, vals_b, False)
        step(2 * p + 1, rows_b, vals_b, rows_a, vals_a, False)

    step(NB2 - 2, rows_a, vals_a, rows_b, vals_b, False)
    step(NB2 - 1, rows_b, vals_b, None, None, True)

    plsc.subcore_barrier()
    pltpu.sync_copy(y_sh.at[pl.ds(sid * EP_T, EP_T)],
                    outy.at[cid, pl.ds(sid * EP_T, EP_T)])

    @pl.when(sid == 0)
    def _():
        pltpu.sync_copy(s_sh, outs.at[pl.ds(cid * EP, EP)])


def _v2e(xt, src3, dst3, dv1, zr, zs):
    k = pl.kernel(
        _v2e_body,
        out_type=(jax.ShapeDtypeStruct((NC, EP, D), f32),
                  jax.ShapeDtypeStruct((NC * EP,), f32)),
        mesh=_sc_mesh(),
        scratch_types=[
            pltpu.VMEM((NB2, G2), jnp.int32),
            pltpu.VMEM((NB2, G2), jnp.int32),
            pltpu.VMEM((G2, D), f32),
            pltpu.VMEM((G2, D), f32),
            pltpu.VMEM((G2,), f32),
            pltpu.VMEM((G2,), f32),
            pltpu.VMEM_SHARED((EP, D), f32),
            pltpu.VMEM_SHARED((EP,), f32),
            pltpu.SemaphoreType.DMA,
            pltpu.SemaphoreType.DMA,
        ],
    )
    return k(xt, src3, dst3, dv1, zr, zs)


G2 = 128
NNZP = NW * VP
NB2 = VP // G2
NCH = 8
CB = 10


def _e2v_body(y_hbm, src_hbm, dst_hbm, zr_hbm, outx,
              idx_s, idx_d, rows_a, rows_b, x_sh, sg):
    cid = lax.axis_index("c")
    sid = lax.axis_index("s")
    wid = cid * NS + sid
    pltpu.sync_copy(zr_hbm, x_sh.at[pl.ds(sid * VP_T, VP_T)])
    plsc.subcore_barrier()

    @pl.loop(0, NCH)
    def _(c):
        pltpu.sync_copy(src_hbm.at[wid, c], idx_s)
        pltpu.sync_copy(dst_hbm.at[wid, c], idx_d)
        pltpu.async_copy(y_hbm.at[idx_d.at[0]], rows_a, sg)

        def step(g, rcur, rnxt, last):
            pltpu.make_async_copy(y_hbm.at[idx_d.at[g]], rcur, sg).wait()
            if not last:
                pltpu.async_copy(y_hbm.at[idx_d.at[g + 1]], rnxt, sg)
            pltpu.sync_copy(rcur, x_sh.at[idx_s.at[g]], add=True)

        @pl.loop(0, (CB - 2) // 2)
        def _(p):
            step(2 * p, rows_a, rows_b, False)
            step(2 * p + 1, rows_b, rows_a, False)

        step(CB - 2, rows_a, rows_b, False)
        step(CB - 1, rows_b, None, True)

    plsc.subcore_barrier()
    pltpu.sync_copy(x_sh.at[pl.ds(sid * VP_T, VP_T)],
                    outx.at[cid, pl.ds(sid * VP_T, VP_T)])


def _e2v(ynorm, src4, dst4, zrv):
    k = pl.kernel(
        _e2v_body,
        out_type=jax.ShapeDtypeStruct((NC, VP, D), f32),
        mesh=_sc_mesh(),
        scratch_types=[
            pltpu.VMEM((CB, G2), jnp.int32),
            pltpu.VMEM((CB, G2), jnp.int32),
            pltpu.VMEM((G2, D), f32),
            pltpu.VMEM((G2, D), f32),
            pltpu.VMEM_SHARED((VP, D), f32),
            pltpu.SemaphoreType.DMA,
        ],
    )
    return k(ynorm, src4, dst4, zrv)


def _matmul_body(x_ref, w_ref, b_ref, o_ref):
    o_ref[...] = lax.dot_general(
        x_ref[...], w_ref[...], (((1,), (1,)), ((), ())),
        preferred_element_type=f32) + b_ref[...]


def _matmul(X, W, b):
    return pl.pallas_call(
        _matmul_body,
        grid=(10,),
        in_specs=[
            pl.BlockSpec((V // 10, D), lambda i: (i, 0)),
            pl.BlockSpec((D, D), lambda i: (0, 0)),
            pl.BlockSpec((1, D), lambda i: (0, 0)),
        ],
        out_specs=pl.BlockSpec((V // 10, D), lambda i: (i, 0)),
        out_shape=jax.ShapeDtypeStruct((V, D), f32),
    )(X, W, b.reshape(1, D))


def _combine_body(d0, d1, c0, c1, dv, dvn, cnt):
    dsum = d0[...] + d1[...]
    dv[...] = dsum
    dvn[...] = jnp.where(dsum > 0, lax.rsqrt(jnp.maximum(dsum, 1e-12)), 0.0)
    cnt[...] = c0[...] + c1[...]


def _combine(d0, d1, c0, c1):
    return pl.pallas_call(
        _combine_body,
        out_shape=(jax.ShapeDtypeStruct((VP // D, D), f32),
                   jax.ShapeDtypeStruct((VP // D, D), f32),
                   jax.ShapeDtypeStruct((EP // D, D), f32)),
    )(d0, d1, c0, c1)


def _norm_body(y0, y1, s0, s1, c, out):
    cc = c[...]
    y = (y0[...] + y1[...]) / jnp.maximum(cc, 1.0)
    de = (s0[...] + s1[...]) / (cc + 1.0)
    fac = jnp.where(cc > 0, lax.rsqrt(jnp.maximum(de, 1e-12)), 1.0)
    out[...] = y * fac


def _norm(y0, y1, s0, s1, cnt):
    nb = 8
    return pl.pallas_call(
        _norm_body,
        grid=(nb * YREP,),
        in_specs=[
            pl.BlockSpec((EP // nb, D), lambda i: (i % nb, 0)),
            pl.BlockSpec((EP // nb, D), lambda i: (i % nb, 0)),
            pl.BlockSpec((EP // nb, 1), lambda i: (i % nb, 0)),
            pl.BlockSpec((EP // nb, 1), lambda i: (i % nb, 0)),
            pl.BlockSpec((EP // nb, 1), lambda i: (i % nb, 0)),
        ],
        out_specs=pl.BlockSpec((EP // nb, D), lambda i: (i, 0)),
        out_shape=jax.ShapeDtypeStruct((YREP * EP, D), f32),
    )(y0, y1, s0, s1, cnt)


def _final_body(x0, x1, dvn, out):
    out[...] = jnp.maximum(dvn[...] * (x0[...] + x1[...]), 0.0)


def _final(x0, x1, dvn):
    nb = 10
    return pl.pallas_call(
        _final_body,
        grid=(nb,),
        in_specs=[
            pl.BlockSpec((V // nb, D), lambda i: (i, 0)),
            pl.BlockSpec((V // nb, D), lambda i: (i, 0)),
            pl.BlockSpec((V // nb, 1), lambda i: (i, 0)),
        ],
        out_specs=pl.BlockSpec((V // nb, D), lambda i: (i, 0)),
        out_shape=jax.ShapeDtypeStruct((V, D), f32),
    )(x0, x1, dvn)


def kernel(X, v2e_src, v2e_dst, W, b):
    npad = NNZP - NNZ
    iota_p = jnp.arange(npad, dtype=jnp.int32)
    srcpd = jnp.concatenate([v2e_src, (iota_p % (VP - V)) + V])
    dstp0 = jnp.concatenate([v2e_dst, (iota_p % (EP - E)) + E])

    xt = _matmul(X, W, b)

    zv = jnp.zeros((VP_T,), f32)
    ze = jnp.zeros((EP,), f32)
    dvp, cep = _hist(srcpd.reshape(NW, NB2, G2), dstp0.reshape(NW, NB2, G2),
                     zv, ze)
    dvp = dvp.reshape(NC, VP)
    cep = cep.reshape(NC, EP)

    dv, dvneg, cnt = _combine(
        dvp[0].reshape(VP // D, D), dvp[1].reshape(VP // D, D),
        cep[0].reshape(EP // D, D), cep[1].reshape(EP // D, D))
    dv1 = dv.reshape(VP)

    def _wtr2(a):
        return a.reshape(NW, NB2, G2).transpose(0, 2, 1).reshape(NW, NB2, G2)

    srcp2 = _wtr2(jnp.concatenate([v2e_src, iota_p % V]))
    dstp2 = _wtr2(jnp.concatenate([v2e_dst, (iota_p % (EP - E)) + E]))
    zr = jnp.zeros((EP_T, D), f32)
    zs = jnp.zeros((EP,), f32)
    yp, sp = _v2e(xt, srcp2, dstp2, dv1, zr, zs)
    sp = sp.reshape(NC, EP)

    ynorm = _norm(yp[0], yp[1],
                  sp[0].reshape(EP, 1), sp[1].reshape(EP, 1),
                  cnt.reshape(EP, 1))

    def _wtr(a):
        return a.reshape(NW, NB2, G2).transpose(0, 2, 1).reshape(-1)

    dstp = _wtr(dstp0) + (jnp.arange(NNZP, dtype=jnp.int32) % YREP) * EP
    zrv = jnp.zeros((VP_T, D), f32)
    xp = _e2v(ynorm, _wtr(srcpd).reshape(NW, NCH, CB, G2),
              dstp.reshape(NW, NCH, CB, G2), zrv)

    return _final(xp[0], xp[1], dvneg.reshape(VP, 1)[:V])
